# Initial kernel scaffold; baseline (speedup 1.0000x reference)
import functools

import jax
import jax.numpy as jnp
from jax import lax
from jax.experimental import pallas as pl
from jax.experimental.pallas import tpu as pltpu
from jax.experimental.pallas import tpu_sc as plsc

N_NODES = 10000
D = 128
E = 320000

NC = 2
NS = 16
NW = NC * NS
BATCH = 128
EDGES_PER_TILE = E // NW
NBATCH = -(-EDGES_PER_TILE // BATCH)
E_PAD = NW * NBATCH * BATCH
NPAD = 10016
ROWS_PER_TILE = NPAD // NS
DEG_W = 8


def _sc_scatter_body(with_deg, *refs):
    if with_deg:
        (table, src_i, dst_i, zrows, zdeg, ones_h,
         acc_out, deg_out,
         idx_s, idx_d, rows, ones_v, acc, dega, sem) = refs
    else:
        (table, src_i, dst_i, zrows,
         acc_out,
         idx_s, idx_d, rows, acc, sem) = refs
    cid = lax.axis_index("c")
    sid = lax.axis_index("s")
    wid = cid * NS + sid
    pltpu.sync_copy(zrows, acc.at[pl.ds(sid * ROWS_PER_TILE, ROWS_PER_TILE)])
    if with_deg:
        pltpu.sync_copy(zdeg, dega.at[pl.ds(sid * ROWS_PER_TILE, ROWS_PER_TILE)])
        pltpu.sync_copy(ones_h, ones_v)
    pltpu.sync_copy(src_i.at[wid], idx_s)
    pltpu.sync_copy(dst_i.at[wid], idx_d)
    plsc.subcore_barrier()

    def step(j, carry):
        pltpu.async_copy(table.at[idx_s.at[j]], rows, sem).wait()
        pltpu.sync_copy(rows, acc.at[idx_d.at[j]], add=True)
        if with_deg:
            pltpu.sync_copy(ones_v, dega.at[idx_d.at[j]], add=True)
        return carry

    lax.fori_loop(0, NBATCH, step, 0)
    plsc.subcore_barrier()
    pltpu.sync_copy(acc.at[pl.ds(sid * ROWS_PER_TILE, ROWS_PER_TILE)],
                    acc_out.at[cid, pl.ds(sid * ROWS_PER_TILE, ROWS_PER_TILE)])
    if with_deg:
        pltpu.sync_copy(dega.at[pl.ds(sid * ROWS_PER_TILE, ROWS_PER_TILE)],
                        deg_out.at[cid, pl.ds(sid * ROWS_PER_TILE, ROWS_PER_TILE)])


def _make_sc_pass(with_deg):
    mesh = plsc.VectorSubcoreMesh(core_axis_name="c", subcore_axis_name="s")
    out_type = [jax.ShapeDtypeStruct((NC, NPAD, D), jnp.float32)]
    scratch = [
        pltpu.VMEM((NBATCH, BATCH), jnp.int32),
        pltpu.VMEM((NBATCH, BATCH), jnp.int32),
        pltpu.VMEM((BATCH, D), jnp.float32),
    ]
    if with_deg:
        out_type.append(jax.ShapeDtypeStruct((NC, NPAD, DEG_W), jnp.float32))
        scratch.append(pltpu.VMEM((BATCH, DEG_W), jnp.float32))
    scratch.append(pltpu.VMEM_SHARED((NPAD, D), jnp.float32))
    if with_deg:
        scratch.append(pltpu.VMEM_SHARED((NPAD, DEG_W), jnp.float32))
    scratch.append(pltpu.SemaphoreType.DMA)
    return pl.kernel(
        functools.partial(_sc_scatter_body, with_deg),
        mesh=mesh,
        out_type=out_type if len(out_type) > 1 else out_type[0],
        scratch_types=scratch,
    )


def _tc_a_body(x, a0, a1, d0, d1, Wf, bf, Wg1, Wg2, xws_o, self_o):
    agg = a0[...] + a1[...]
    h1 = jnp.dot(agg, Wf[...], preferred_element_type=jnp.float32) + bf[...]
    xw = (jnp.dot(x[...], Wg1[...], preferred_element_type=jnp.float32)
          + jnp.dot(h1, Wg2[...], preferred_element_type=jnp.float32))
    deg = d0[:, :1] + d1[:, :1] + 1.0
    dinv = lax.rsqrt(deg)
    xws_o[...] = xw * dinv
    self_o[...] = xw * (1.0 / deg)


def _tc_b_body(s0, s1, d0, d1, selfc, bg, Wd1, bd1, Wd2, bd2, z_o):
    deg = d0[:, :1] + d1[:, :1] + 1.0
    dinv = lax.rsqrt(deg)
    out = (s0[...] + s1[...]) * dinv + selfc[...] + bg[...]
    h = jnp.maximum(jnp.dot(out, Wd1[...], preferred_element_type=jnp.float32)
                    + bd1[...], 0.0)
    z_o[...] = jnp.dot(h, Wd2[...], preferred_element_type=jnp.float32) + bd2[...]


_ROW_BLK = 1000
_GRID = N_NODES // _ROW_BLK


def _row_spec(w):
    return pl.BlockSpec((_ROW_BLK, w), lambda i: (i, 0))


def _full_spec(h, w):
    return pl.BlockSpec((h, w), lambda i: (0, 0))


def kernel(x, edge_index, Wf, bf, Wg, bg, Wd1, bd1, Wd2, bd2):
    src = edge_index[0].astype(jnp.int32)
    dst = edge_index[1].astype(jnp.int32)
    pad = E_PAD - E
    src_p = jnp.concatenate([src, jnp.zeros((pad,), jnp.int32)])
    dst_p = jnp.concatenate([dst, jnp.full((pad,), N_NODES, jnp.int32)])
    src_t = src_p.reshape(NW, NBATCH, BATCH)
    dst_t = dst_p.reshape(NW, NBATCH, BATCH)

    zrows = jnp.zeros((ROWS_PER_TILE, D), jnp.float32)
    zdeg = jnp.zeros((ROWS_PER_TILE, DEG_W), jnp.float32)
    ones_h = jnp.ones((BATCH, DEG_W), jnp.float32)

    sc_pass1 = _make_sc_pass(True)
    sc_pass2 = _make_sc_pass(False)

    agg_p, deg_p = sc_pass1(x, src_t, dst_t, zrows, zdeg, ones_h)

    a0 = agg_p[0, :N_NODES]
    a1 = agg_p[1, :N_NODES]
    d0 = deg_p[0, :N_NODES]
    d1 = deg_p[1, :N_NODES]

    Wg1 = Wg[:D]
    Wg2 = Wg[D:]
    bf2 = bf.reshape(1, D)
    xws, selfc = pl.pallas_call(
        _tc_a_body,
        grid=(_GRID,),
        in_specs=[
            _row_spec(D), _row_spec(D), _row_spec(D),
            _row_spec(DEG_W), _row_spec(DEG_W),
            _full_spec(D, D), _full_spec(1, D),
            _full_spec(D, D), _full_spec(D, D),
        ],
        out_specs=[_row_spec(D), _row_spec(D)],
        out_shape=[
            jax.ShapeDtypeStruct((N_NODES, D), jnp.float32),
            jax.ShapeDtypeStruct((N_NODES, D), jnp.float32),
        ],
    )(x, a0, a1, d0, d1, Wf, bf2, Wg1, Wg2)

    s_p = sc_pass2(xws, src_t, dst_t, zrows)

    HID2 = Wd1.shape[1]
    Wd2p = jnp.zeros((HID2, D), jnp.float32).at[:, :2].set(Wd2)
    bd2p = jnp.zeros((1, D), jnp.float32).at[:, :2].set(bd2)
    zpad = pl.pallas_call(
        _tc_b_body,
        grid=(_GRID,),
        in_specs=[
            _row_spec(D), _row_spec(D),
            _row_spec(DEG_W), _row_spec(DEG_W),
            _row_spec(D), _full_spec(1, D),
            _full_spec(D, HID2), _full_spec(1, HID2),
            _full_spec(HID2, D), _full_spec(1, D),
        ],
        out_specs=_row_spec(D),
        out_shape=jax.ShapeDtypeStruct((N_NODES, D), jnp.float32),
    )(s_p[0, :N_NODES], s_p[1, :N_NODES], d0, d1, selfc,
      bg.reshape(1, D), Wd1, bd1.reshape(1, HID2), Wd2p, bd2p)

    return zpad[:, :2]


# speedup vs baseline: 6.7122x; 6.7122x over previous
import functools

import jax
import jax.numpy as jnp
from jax import lax
from jax.experimental import pallas as pl
from jax.experimental.pallas import tpu as pltpu
from jax.experimental.pallas import tpu_sc as plsc

N_NODES = 10000
D = 128
E = 320000

NC = 2
NS = 16
NW = NC * NS
BATCH = 128
NBATCH = 80
CH = 8
E_PAD = NW * NBATCH * BATCH
NPAD = 10112
ROWS_PER_TILE = NPAD // NS
_CHUNKS = [(0, 128), (128, 128), (256, 128), (384, 128), (512, 120)]


def _sc_feat_body(table, src_i, dst_i, zrows,
                  acc_out,
                  idx_s, idx_d, rows, acc, sem):
    cid = lax.axis_index("c")
    sid = lax.axis_index("s")
    wid = cid * NS + sid
    base = sid * ROWS_PER_TILE
    pltpu.sync_copy(zrows, rows)
    for k, sz in _CHUNKS:
        pltpu.sync_copy(rows.at[pl.ds(0, sz)], acc.at[pl.ds(base + k, sz)])
    plsc.subcore_barrier()

    def chunk(c, carry):
        pltpu.sync_copy(src_i.at[wid, pl.ds(c * CH, CH)], idx_s)
        pltpu.sync_copy(dst_i.at[wid, pl.ds(c * CH, CH)], idx_d)

        def step(j, carry2):
            pltpu.async_copy(table.at[idx_s.at[j]], rows, sem).wait()
            pltpu.sync_copy(rows, acc.at[idx_d.at[j]], add=True)
            return carry2

        return lax.fori_loop(0, CH, step, carry)

    lax.fori_loop(0, NBATCH // CH, chunk, 0)
    plsc.subcore_barrier()
    for k, sz in _CHUNKS:
        pltpu.sync_copy(acc.at[pl.ds(base + k, sz)], rows.at[pl.ds(0, sz)])
        pltpu.sync_copy(rows.at[pl.ds(0, sz)],
                        acc_out.at[cid, pl.ds(base + k, sz)])


def _sc_deg_body(dst_i, ones_h,
                 deg_out,
                 idx_d, rows, acc, sem):
    del sem
    cid = lax.axis_index("c")
    sid = lax.axis_index("s")
    wid = cid * NS + sid
    base = sid * ROWS_PER_TILE
    pltpu.sync_copy(ones_h.at[1], rows)
    for k, sz in _CHUNKS:
        pltpu.sync_copy(rows.at[pl.ds(0, sz)], acc.at[pl.ds(base + k, sz)])
    pltpu.sync_copy(ones_h.at[0], rows)
    plsc.subcore_barrier()

    def chunk(c, carry):
        pltpu.sync_copy(dst_i.at[wid, pl.ds(c * CH, CH)], idx_d)

        def step(j, carry2):
            pltpu.sync_copy(rows, acc.at[idx_d.at[j]], add=True)
            return carry2

        return lax.fori_loop(0, CH, step, carry)

    lax.fori_loop(0, NBATCH // CH, chunk, 0)
    plsc.subcore_barrier()
    for k, sz in _CHUNKS:
        pltpu.sync_copy(acc.at[pl.ds(base + k, sz)], rows.at[pl.ds(0, sz)])
        pltpu.sync_copy(rows.at[pl.ds(0, sz)],
                        deg_out.at[cid, pl.ds(base + k, sz)])


_MESH = dict(core_axis_name="c", subcore_axis_name="s",
             num_cores=NC, num_subcores=NS)


def _make_feat_pass():
    return pl.kernel(
        _sc_feat_body,
        mesh=plsc.VectorSubcoreMesh(**_MESH),
        out_type=jax.ShapeDtypeStruct((NC, NPAD, D), jnp.float32),
        scratch_types=[
            pltpu.VMEM((CH, BATCH), jnp.int32),
            pltpu.VMEM((CH, BATCH), jnp.int32),
            pltpu.VMEM((BATCH, D), jnp.float32),
            pltpu.VMEM_SHARED((NPAD, D), jnp.float32),
            pltpu.SemaphoreType.DMA,
        ],
    )


def _make_deg_pass():
    return pl.kernel(
        _sc_deg_body,
        mesh=plsc.VectorSubcoreMesh(**_MESH),
        out_type=jax.ShapeDtypeStruct((NC, NPAD, D), jnp.float32),
        scratch_types=[
            pltpu.VMEM((CH, BATCH), jnp.int32),
            pltpu.VMEM((BATCH, D), jnp.float32),
            pltpu.VMEM_SHARED((NPAD, D), jnp.float32),
            pltpu.SemaphoreType.DMA,
        ],
    )


def _tc_a_body(x, a0, a1, d0, d1, Wf, bf, Wg1, Wg2, xws_o, self_o):
    agg = a0[...] + a1[...]
    h1 = jnp.dot(agg, Wf[...], preferred_element_type=jnp.float32) + bf[...]
    xw = (jnp.dot(x[...], Wg1[...], preferred_element_type=jnp.float32)
          + jnp.dot(h1, Wg2[...], preferred_element_type=jnp.float32))
    deg = d0[:, :1] + d1[:, :1] + 1.0
    dinv = lax.rsqrt(deg)
    xws_o[...] = xw * dinv
    self_o[...] = xw * (1.0 / deg)


def _tc_b_body(s0, s1, d0, d1, selfc, bg, Wd1, bd1, Wd2, bd2, z_o):
    deg = d0[:, :1] + d1[:, :1] + 1.0
    dinv = lax.rsqrt(deg)
    out = (s0[...] + s1[...]) * dinv + selfc[...] + bg[...]
    h = jnp.maximum(jnp.dot(out, Wd1[...], preferred_element_type=jnp.float32)
                    + bd1[...], 0.0)
    z_o[...] = jnp.dot(h, Wd2[...], preferred_element_type=jnp.float32) + bd2[...]


_ROW_BLK = 1000
_GRID = N_NODES // _ROW_BLK


def _row_spec(w):
    return pl.BlockSpec((_ROW_BLK, w), lambda i: (i, 0))


def _deg_spec():
    return pl.BlockSpec((_ROW_BLK, D), lambda i: (i, 0))


def _full_spec(h, w):
    return pl.BlockSpec((h, w), lambda i: (0, 0))


def kernel(x, edge_index, Wf, bf, Wg, bg, Wd1, bd1, Wd2, bd2):
    src = edge_index[0].astype(jnp.int32)
    dst = edge_index[1].astype(jnp.int32)
    pad = E_PAD - E
    src_p = jnp.concatenate([src, jnp.zeros((pad,), jnp.int32)])
    dst_p = jnp.concatenate([dst, jnp.full((pad,), N_NODES, jnp.int32)])
    src_t = src_p.reshape(NW, NBATCH, BATCH)
    dst_t = dst_p.reshape(NW, NBATCH, BATCH)

    zrows = jnp.zeros((BATCH, D), jnp.float32)
    ones_h = jnp.stack([jnp.ones((BATCH, D), jnp.float32), zrows])

    feat_pass = _make_feat_pass()
    deg_pass = _make_deg_pass()

    agg_p = feat_pass(x, src_t, dst_t, zrows)
    deg_p = deg_pass(dst_t, ones_h)

    a0 = agg_p[0, :N_NODES]
    a1 = agg_p[1, :N_NODES]
    d0 = deg_p[0, :N_NODES]
    d1 = deg_p[1, :N_NODES]

    Wg1 = Wg[:D]
    Wg2 = Wg[D:]
    bf2 = bf.reshape(1, D)
    xws, selfc = pl.pallas_call(
        _tc_a_body,
        grid=(_GRID,),
        in_specs=[
            _row_spec(D), _row_spec(D), _row_spec(D),
            _deg_spec(), _deg_spec(),
            _full_spec(D, D), _full_spec(1, D),
            _full_spec(D, D), _full_spec(D, D),
        ],
        out_specs=[_row_spec(D), _row_spec(D)],
        out_shape=[
            jax.ShapeDtypeStruct((N_NODES, D), jnp.float32),
            jax.ShapeDtypeStruct((N_NODES, D), jnp.float32),
        ],
    )(x, a0, a1, d0, d1, Wf, bf2, Wg1, Wg2)

    s_p = feat_pass(xws, src_t, dst_t, zrows)

    HID2 = Wd1.shape[1]
    Wd2p = jnp.zeros((HID2, D), jnp.float32).at[:, :2].set(Wd2)
    bd2p = jnp.zeros((1, D), jnp.float32).at[:, :2].set(bd2)
    zpad = pl.pallas_call(
        _tc_b_body,
        grid=(_GRID,),
        in_specs=[
            _row_spec(D), _row_spec(D),
            _deg_spec(), _deg_spec(),
            _row_spec(D), _full_spec(1, D),
            _full_spec(D, HID2), _full_spec(1, HID2),
            _full_spec(HID2, D), _full_spec(1, D),
        ],
        out_specs=_row_spec(D),
        out_shape=jax.ShapeDtypeStruct((N_NODES, D), jnp.float32),
    )(s_p[0, :N_NODES], s_p[1, :N_NODES], d0, d1, selfc,
      bg.reshape(1, D), Wd1, bd1.reshape(1, HID2), Wd2p, bd2p)

    return zpad[:, :2]


# speedup vs baseline: 7.3002x; 1.0876x over previous
import functools

import jax
import jax.numpy as jnp
from jax import lax
from jax.experimental import pallas as pl
from jax.experimental.pallas import tpu as pltpu
from jax.experimental.pallas import tpu_sc as plsc

N_NODES = 10000
D = 128
E = 320000

NC = 2
NS = 16
NW = NC * NS
BATCH = 128
NBATCH = 80
CH = 8
E_PAD = NW * NBATCH * BATCH
NPAD = 10112
ROWS_PER_TILE = NPAD // NS
_CHUNKS = [(0, 128), (128, 128), (256, 128), (384, 128), (512, 120)]


def _sc_feat_body(table, src_i, dst_i, zrows,
                  acc_out,
                  idx_s, idx_d, rows0, rows1, sem0, sem1, acc):
    cid = lax.axis_index("c")
    sid = lax.axis_index("s")
    wid = cid * NS + sid
    base = sid * ROWS_PER_TILE
    bufs = (rows0, rows1)
    sems = (sem0, sem1)
    pltpu.sync_copy(zrows, rows0)
    for k, sz in _CHUNKS:
        pltpu.sync_copy(rows0.at[pl.ds(0, sz)], acc.at[pl.ds(base + k, sz)])
    plsc.subcore_barrier()

    def chunk(c, carry):
        pltpu.sync_copy(src_i.at[wid, pl.ds(c * CH, CH)], idx_s)
        pltpu.sync_copy(dst_i.at[wid, pl.ds(c * CH, CH)], idx_d)
        descs = [None, None]
        descs[0] = pltpu.async_copy(table.at[idx_s.at[0]], bufs[0], sems[0])
        for j in range(CH):
            if j + 1 < CH:
                b = (j + 1) % 2
                descs[b] = pltpu.async_copy(table.at[idx_s.at[j + 1]],
                                            bufs[b], sems[b])
            descs[j % 2].wait()
            pltpu.sync_copy(bufs[j % 2], acc.at[idx_d.at[j]], add=True)
        return carry

    lax.fori_loop(0, NBATCH // CH, chunk, 0)
    plsc.subcore_barrier()
    for k, sz in _CHUNKS:
        pltpu.sync_copy(acc.at[pl.ds(base + k, sz)], rows0.at[pl.ds(0, sz)])
        pltpu.sync_copy(rows0.at[pl.ds(0, sz)],
                        acc_out.at[cid, pl.ds(base + k, sz)])


def _sc_deg_body(dst_i, ones_h,
                 deg_out,
                 idx_d, rows, acc, sem):
    del sem
    cid = lax.axis_index("c")
    sid = lax.axis_index("s")
    wid = cid * NS + sid
    base = sid * ROWS_PER_TILE
    pltpu.sync_copy(ones_h.at[1], rows)
    for k, sz in _CHUNKS:
        pltpu.sync_copy(rows.at[pl.ds(0, sz)], acc.at[pl.ds(base + k, sz)])
    pltpu.sync_copy(ones_h.at[0], rows)
    plsc.subcore_barrier()

    def chunk(c, carry):
        pltpu.sync_copy(dst_i.at[wid, pl.ds(c * CH, CH)], idx_d)

        def step(j, carry2):
            pltpu.sync_copy(rows, acc.at[idx_d.at[j]], add=True)
            return carry2

        return lax.fori_loop(0, CH, step, carry)

    lax.fori_loop(0, NBATCH // CH, chunk, 0)
    plsc.subcore_barrier()
    for k, sz in _CHUNKS:
        pltpu.sync_copy(acc.at[pl.ds(base + k, sz)], rows.at[pl.ds(0, sz)])
        pltpu.sync_copy(rows.at[pl.ds(0, sz)],
                        deg_out.at[cid, pl.ds(base + k, sz)])


_MESH = dict(core_axis_name="c", subcore_axis_name="s",
             num_cores=NC, num_subcores=NS)


def _make_feat_pass():
    return pl.kernel(
        _sc_feat_body,
        mesh=plsc.VectorSubcoreMesh(**_MESH),
        out_type=jax.ShapeDtypeStruct((NC, NPAD, D), jnp.float32),
        scratch_types=[
            pltpu.VMEM((CH, BATCH), jnp.int32),
            pltpu.VMEM((CH, BATCH), jnp.int32),
            pltpu.VMEM((BATCH, D), jnp.float32),
            pltpu.VMEM((BATCH, D), jnp.float32),
            pltpu.SemaphoreType.DMA,
            pltpu.SemaphoreType.DMA,
            pltpu.VMEM_SHARED((NPAD, D), jnp.float32),
        ],
    )


def _make_deg_pass():
    return pl.kernel(
        _sc_deg_body,
        mesh=plsc.VectorSubcoreMesh(**_MESH),
        out_type=jax.ShapeDtypeStruct((NC, NPAD, D), jnp.float32),
        scratch_types=[
            pltpu.VMEM((CH, BATCH), jnp.int32),
            pltpu.VMEM((BATCH, D), jnp.float32),
            pltpu.VMEM_SHARED((NPAD, D), jnp.float32),
            pltpu.SemaphoreType.DMA,
        ],
    )


def _tc_a_body(x, a0, a1, d0, d1, Wf, bf, Wg1, Wg2, xws_o, self_o):
    agg = a0[...] + a1[...]
    h1 = jnp.dot(agg, Wf[...], preferred_element_type=jnp.float32) + bf[...]
    xw = (jnp.dot(x[...], Wg1[...], preferred_element_type=jnp.float32)
          + jnp.dot(h1, Wg2[...], preferred_element_type=jnp.float32))
    deg = d0[:, :1] + d1[:, :1] + 1.0
    dinv = lax.rsqrt(deg)
    xws_o[...] = xw * dinv
    self_o[...] = xw * (1.0 / deg)


def _tc_b_body(s0, s1, d0, d1, selfc, bg, Wd1, bd1, Wd2, bd2, z_o):
    deg = d0[:, :1] + d1[:, :1] + 1.0
    dinv = lax.rsqrt(deg)
    out = (s0[...] + s1[...]) * dinv + selfc[...] + bg[...]
    h = jnp.maximum(jnp.dot(out, Wd1[...], preferred_element_type=jnp.float32)
                    + bd1[...], 0.0)
    z_o[...] = jnp.dot(h, Wd2[...], preferred_element_type=jnp.float32) + bd2[...]


_ROW_BLK = 1000
_GRID = N_NODES // _ROW_BLK


def _row_spec(w):
    return pl.BlockSpec((_ROW_BLK, w), lambda i: (i, 0))


def _deg_spec():
    return pl.BlockSpec((_ROW_BLK, D), lambda i: (i, 0))


def _full_spec(h, w):
    return pl.BlockSpec((h, w), lambda i: (0, 0))


def kernel(x, edge_index, Wf, bf, Wg, bg, Wd1, bd1, Wd2, bd2):
    src = edge_index[0].astype(jnp.int32)
    dst = edge_index[1].astype(jnp.int32)
    pad = E_PAD - E
    src_p = jnp.concatenate([src, jnp.zeros((pad,), jnp.int32)])
    dst_p = jnp.concatenate([dst, jnp.full((pad,), N_NODES, jnp.int32)])
    src_t = src_p.reshape(NW, NBATCH, BATCH)
    dst_t = dst_p.reshape(NW, NBATCH, BATCH)

    zrows = jnp.zeros((BATCH, D), jnp.float32)
    ones_h = jnp.stack([jnp.ones((BATCH, D), jnp.float32), zrows])

    feat_pass = _make_feat_pass()
    deg_pass = _make_deg_pass()

    agg_p = feat_pass(x, src_t, dst_t, zrows)
    deg_p = deg_pass(dst_t, ones_h)

    a0 = agg_p[0, :N_NODES]
    a1 = agg_p[1, :N_NODES]
    d0 = deg_p[0, :N_NODES]
    d1 = deg_p[1, :N_NODES]

    Wg1 = Wg[:D]
    Wg2 = Wg[D:]
    bf2 = bf.reshape(1, D)
    xws, selfc = pl.pallas_call(
        _tc_a_body,
        grid=(_GRID,),
        in_specs=[
            _row_spec(D), _row_spec(D), _row_spec(D),
            _deg_spec(), _deg_spec(),
            _full_spec(D, D), _full_spec(1, D),
            _full_spec(D, D), _full_spec(D, D),
        ],
        out_specs=[_row_spec(D), _row_spec(D)],
        out_shape=[
            jax.ShapeDtypeStruct((N_NODES, D), jnp.float32),
            jax.ShapeDtypeStruct((N_NODES, D), jnp.float32),
        ],
    )(x, a0, a1, d0, d1, Wf, bf2, Wg1, Wg2)

    s_p = feat_pass(xws, src_t, dst_t, zrows)

    HID2 = Wd1.shape[1]
    Wd2p = jnp.zeros((HID2, D), jnp.float32).at[:, :2].set(Wd2)
    bd2p = jnp.zeros((1, D), jnp.float32).at[:, :2].set(bd2)
    zpad = pl.pallas_call(
        _tc_b_body,
        grid=(_GRID,),
        in_specs=[
            _row_spec(D), _row_spec(D),
            _deg_spec(), _deg_spec(),
            _row_spec(D), _full_spec(1, D),
            _full_spec(D, HID2), _full_spec(1, HID2),
            _full_spec(HID2, D), _full_spec(1, D),
        ],
        out_specs=_row_spec(D),
        out_shape=jax.ShapeDtypeStruct((N_NODES, D), jnp.float32),
    )(s_p[0, :N_NODES], s_p[1, :N_NODES], d0, d1, selfc,
      bg.reshape(1, D), Wd1, bd1.reshape(1, HID2), Wd2p, bd2p)

    return zpad[:, :2]
